# baseline (speedup 1.0000x reference)
import functools

import jax
import jax.numpy as jnp
from jax import lax
from jax.experimental import pallas as pl
from jax.experimental.pallas import tpu as pltpu
from jax.experimental.pallas import tpu_sc as plsc


def _table_body(beta_ref, alpha_ref, mu_ref, obs_ref, t_ref, s_ref,
                out_ref, idx_ref):
    n_time, n_space = obs_ref.shape
    beta = beta_ref[0, 0]
    ti = lax.broadcasted_iota(jnp.int32, (n_time, n_time), 0)
    tp = lax.broadcasted_iota(jnp.int32, (n_time, n_time), 1)
    w = jnp.where(tp < ti, jnp.exp(-beta * (ti - tp).astype(jnp.float32)), 0.0)
    d = jnp.dot(w, obs_ref[...].astype(jnp.float32),
                preferred_element_type=jnp.float32,
                precision=lax.Precision.HIGHEST)
    ii = lax.broadcasted_iota(jnp.int32, (n_space, n_space), 0)
    jj = lax.broadcasted_iota(jnp.int32, (n_space, n_space), 1)
    adiag = jnp.sum(jnp.where(ii == jj, alpha_ref[...], 0.0),
                    axis=0, keepdims=True)
    out_ref[...] = jnp.maximum(mu_ref[...] + (beta * adiag) * d, 1e-5)
    idx_ref[...] = t_ref[...] * n_space + s_ref[...]


def _build_table(n_time, n_space, batch):
    return pl.pallas_call(
        _table_body,
        out_shape=[
            jax.ShapeDtypeStruct((n_time, n_space), jnp.float32),
            jax.ShapeDtypeStruct((batch,), jnp.int32),
        ],
        in_specs=[
            pl.BlockSpec(memory_space=pltpu.SMEM),
            pl.BlockSpec(memory_space=pltpu.VMEM),
            pl.BlockSpec(memory_space=pltpu.VMEM),
            pl.BlockSpec(memory_space=pltpu.VMEM),
            pl.BlockSpec(memory_space=pltpu.VMEM),
            pl.BlockSpec(memory_space=pltpu.VMEM),
        ],
    )


_NC, _NS, _L = 2, 16, 16


def _build_gather(batch):
    nw = 8
    bpw = batch // nw
    mesh = plsc.VectorSubcoreMesh(core_axis_name="c", subcore_axis_name="s", num_cores=1, num_subcores=8)

    @functools.partial(
        pl.kernel,
        mesh=mesh,
        out_type=jax.ShapeDtypeStruct((batch,), jnp.float32),
        scratch_types=[
            pltpu.VMEM((bpw,), jnp.int32),
            pltpu.VMEM((bpw,), jnp.float32),
            pltpu.SemaphoreType.DMA,
        ],
    )
    def gk(tab_hbm, idx_hbm, out_hbm, idx_v, val_v, sem):
        wid = lax.axis_index("s")
        base = wid * bpw
        pltpu.sync_copy(idx_hbm.at[pl.ds(base, bpw)], idx_v)
        pltpu.async_copy(tab_hbm.at[idx_v], val_v, sem).wait()
        pltpu.sync_copy(val_v, out_hbm.at[pl.ds(base, bpw)])

    return gk


def kernel(alpha, beta, mu, obs, t, s):
    n_time, n_space = obs.shape
    batch = t.shape[0]
    table, idx = _build_table(n_time, n_space, batch)(
        beta.reshape(1, 1), alpha, mu.reshape(1, n_space), obs, t, s)
    return _build_gather(batch)(table.reshape(-1), idx)


# speedup vs baseline: 1.0635x; 1.0635x over previous
import functools

import jax
import jax.numpy as jnp
from jax import lax
from jax.experimental import pallas as pl
from jax.experimental.pallas import tpu as pltpu
from jax.experimental.pallas import tpu_sc as plsc


def _table_body(beta_ref, alpha_ref, mu_ref, obs_ref, t_ref, s_ref,
                out_ref, idx_ref):
    n_time, n_space = obs_ref.shape
    beta = beta_ref[0, 0]
    ti = lax.broadcasted_iota(jnp.int32, (n_time, n_time), 0)
    tp = lax.broadcasted_iota(jnp.int32, (n_time, n_time), 1)
    w = jnp.where(tp < ti, jnp.exp(-beta * (ti - tp).astype(jnp.float32)), 0.0)
    d = jnp.dot(w, obs_ref[...].astype(jnp.float32),
                preferred_element_type=jnp.float32,
                precision=lax.Precision.HIGHEST)
    ii = lax.broadcasted_iota(jnp.int32, (n_space, n_space), 0)
    jj = lax.broadcasted_iota(jnp.int32, (n_space, n_space), 1)
    adiag = jnp.sum(jnp.where(ii == jj, alpha_ref[...], 0.0),
                    axis=0, keepdims=True)
    out_ref[...] = jnp.maximum(mu_ref[...] + (beta * adiag) * d, 1e-5)
    idx_ref[...] = t_ref[...] * n_space + s_ref[...]


def _build_table(n_time, n_space, batch):
    return pl.pallas_call(
        _table_body,
        out_shape=[
            jax.ShapeDtypeStruct((n_time, n_space), jnp.float32),
            jax.ShapeDtypeStruct((batch,), jnp.int32),
        ],
        in_specs=[
            pl.BlockSpec(memory_space=pltpu.SMEM),
            pl.BlockSpec(memory_space=pltpu.VMEM),
            pl.BlockSpec(memory_space=pltpu.VMEM),
            pl.BlockSpec(memory_space=pltpu.VMEM),
            pl.BlockSpec(memory_space=pltpu.VMEM),
            pl.BlockSpec(memory_space=pltpu.VMEM),
        ],
    )


_NC, _NS, _L = 2, 16, 16


def _build_gather(batch):
    nw = _NS
    bpw = batch // nw
    mesh = plsc.VectorSubcoreMesh(core_axis_name="c", subcore_axis_name="s", num_cores=1)

    @functools.partial(
        pl.kernel,
        mesh=mesh,
        out_type=jax.ShapeDtypeStruct((batch,), jnp.float32),
        scratch_types=[
            pltpu.VMEM((bpw,), jnp.int32),
            pltpu.VMEM((bpw,), jnp.float32),
            pltpu.SemaphoreType.DMA,
        ],
    )
    def gk(tab_hbm, idx_hbm, out_hbm, idx_v, val_v, sem):
        wid = lax.axis_index("s")
        base = wid * bpw
        pltpu.sync_copy(idx_hbm.at[pl.ds(base, bpw)], idx_v)
        pltpu.async_copy(tab_hbm.at[idx_v], val_v, sem).wait()
        pltpu.sync_copy(val_v, out_hbm.at[pl.ds(base, bpw)])

    return gk


def kernel(alpha, beta, mu, obs, t, s):
    n_time, n_space = obs.shape
    batch = t.shape[0]
    table, idx = _build_table(n_time, n_space, batch)(
        beta.reshape(1, 1), alpha, mu.reshape(1, n_space), obs, t, s)
    return _build_gather(batch)(table.reshape(-1), idx)
